# Initial kernel scaffold; baseline (speedup 1.0000x reference)
import jax
import jax.numpy as jnp
from jax.experimental import pallas as pl


def kernel(x, edge_index, edge_attr, conv1_W, conv1_b, conv2_W, conv2_b, Wx, bx, Wh, bh, wc, bg, lin_W, lin_b):
    raise NotImplementedError("write your pallas kernel here")


# speedup vs baseline: 20.7956x; 20.7956x over previous
import functools

import jax
import jax.numpy as jnp
from jax import lax
from jax.experimental import pallas as pl
from jax.experimental.pallas import tpu as pltpu
from jax.experimental.pallas import tpu_sc as plsc


def _lane_bcast(v, l):
    idx = jnp.full((16, 1), l, jnp.int32)
    return lax.gather(
        v, idx,
        lax.GatherDimensionNumbers(
            offset_dims=(), collapsed_slice_dims=(0,), start_index_map=(0,)),
        (1,), mode=lax.GatherScatterMode.PROMISE_IN_BOUNDS)

def _make_deg_kernel(NC, NS, NPAD, RPT, CH):
    mesh = plsc.VectorSubcoreMesh(core_axis_name="c", subcore_axis_name="s")

    @functools.partial(
        pl.kernel,
        out_type=jax.ShapeDtypeStruct((NC, NPAD), jnp.float32),
        mesh=mesh,
        scratch_types=[
            pltpu.VMEM((CH, 128), jnp.int32),
            pltpu.VMEM((CH, 128), jnp.float32),
            pltpu.VMEM((RPT,), jnp.float32),
            pltpu.VMEM_SHARED((NPAD,), jnp.float32),
        ],
    )
    def deg_kernel(dsts_hbm, ws_hbm, out_hbm, dst_v, w_v, io_v, acc_sh):
        cid = lax.axis_index("c")
        sid = lax.axis_index("s")
        wid = cid * NS + sid

        pltpu.sync_copy(dsts_hbm.at[wid], dst_v)
        pltpu.sync_copy(ws_hbm.at[wid], w_v)

        def zero_body(r, _):
            io_v[pl.ds(r * 16, 16)] = jnp.zeros((16,), jnp.float32)
            return 0
        lax.fori_loop(0, RPT // 16, zero_body, 0)
        pltpu.sync_copy(io_v, acc_sh.at[pl.ds(sid * RPT, RPT)])
        plsc.subcore_barrier()

        def edge_body(j, _):
            pltpu.sync_copy(w_v.at[j], acc_sh.at[dst_v.at[j]], add=True)
            return 0
        lax.fori_loop(0, CH, edge_body, 0)
        plsc.subcore_barrier()

        pltpu.sync_copy(acc_sh.at[pl.ds(sid * RPT, RPT)], io_v)
        pltpu.sync_copy(io_v, out_hbm.at[cid, pl.ds(sid * RPT, RPT)])

    return deg_kernel


def _make_prop_kernel(NC, NS, NPAD, RPT, CH, D):
    mesh = plsc.VectorSubcoreMesh(core_axis_name="c", subcore_axis_name="s")
    RB = 128
    NB = RPT // RB

    @functools.partial(
        pl.kernel,
        out_type=jax.ShapeDtypeStruct((NC, NPAD, D), jnp.float32),
        mesh=mesh,
        scratch_types=[
            pltpu.VMEM((CH, 128), jnp.int32),
            pltpu.VMEM((CH, 128), jnp.int32),
            pltpu.VMEM((CH, 128), jnp.float32),
            pltpu.VMEM((128, D), jnp.float32),
            pltpu.VMEM((RB, D), jnp.float32),
            pltpu.VMEM_SHARED((NPAD, D), jnp.float32),
            pltpu.SemaphoreType.DMA,
        ],
        compiler_params=pltpu.CompilerParams(use_tc_tiling_on_sc=False),
    )
    def prop_kernel(table_hbm, srcs_hbm, dsts_hbm, ws_hbm, out_hbm,
                    src_v, dst_v, w_v, rows_v, io_v, acc_sh, sem):
        cid = lax.axis_index("c")
        sid = lax.axis_index("s")
        wid = cid * NS + sid

        pltpu.sync_copy(srcs_hbm.at[wid], src_v)
        pltpu.sync_copy(dsts_hbm.at[wid], dst_v)
        pltpu.sync_copy(ws_hbm.at[wid], w_v)

        def zero_body(r, _):
            for k in range(D // 16):
                io_v[r, pl.ds(k * 16, 16)] = jnp.zeros((16,), jnp.float32)
            return 0
        lax.fori_loop(0, RB, zero_body, 0)
        for c in range(NB):
            pltpu.sync_copy(io_v, acc_sh.at[pl.ds(sid * RPT + c * RB, RB)])
        plsc.subcore_barrier()

        def edge_body(j, _):
            pltpu.async_copy(table_hbm.at[src_v.at[j]], rows_v, sem).wait()

            def scale_blk(b, _):
                w16 = w_v[j, pl.ds(b * 16, 16)]

                def lane_body(l, _):
                    e = b * 16 + l
                    wv = _lane_bcast(w16, l)
                    for k in range(D // 16):
                        rows_v[e, pl.ds(k * 16, 16)] = (
                            rows_v[e, pl.ds(k * 16, 16)] * wv)
                    return 0
                lax.fori_loop(0, 16, lane_body, 0)
                return 0
            lax.fori_loop(0, 8, scale_blk, 0)

            pltpu.sync_copy(rows_v, acc_sh.at[dst_v.at[j]], add=True)
            return 0
        lax.fori_loop(0, CH, edge_body, 0)
        plsc.subcore_barrier()

        for c in range(NB):
            pltpu.sync_copy(acc_sh.at[pl.ds(sid * RPT + c * RB, RB)], io_v)
            pltpu.sync_copy(io_v, out_hbm.at[cid, pl.ds(sid * RPT + c * RB, RB)])

    return prop_kernel


def _tc_call(body, out_shapes, *args):
    return pl.pallas_call(
        body,
        out_shape=out_shapes,
    )(*args)


def _stage_a(dga_ref, dgb_ref, x_ref, w1_ref, xs1_ref, dis_ref, dis2_ref):
    degw = dga_ref[...] + dgb_ref[...]
    dis = lax.rsqrt(degw + 1.0)
    dis2 = jnp.where(degw > 0.0, lax.rsqrt(jnp.maximum(degw, 1e-30)), 0.0)
    xw1 = jnp.dot(x_ref[...], w1_ref[...], preferred_element_type=jnp.float32)
    xs1_ref[...] = dis * xw1
    dis_ref[...] = dis
    dis2_ref[...] = dis2


def _stage_b(s1a_ref, s1b_ref, xs1_ref, dis_ref, w2_ref, b1_ref, xs2_ref):
    h1 = jnp.maximum(
        dis_ref[...] * (s1a_ref[...] + s1b_ref[...] + xs1_ref[...])
        + b1_ref[...], 0.0)
    xs2_ref[...] = dis_ref[...] * jnp.dot(
        h1, w2_ref[...], preferred_element_type=jnp.float32)


def _stage_c(s2a_ref, s2b_ref, xs2_ref, dis_ref, dis2_ref, b2_ref,
             h2_ref, xs3_ref):
    h2 = jnp.maximum(
        dis_ref[...] * (s2a_ref[...] + s2b_ref[...] + xs2_ref[...])
        + b2_ref[...], 0.0)
    h2_ref[...] = h2
    xs3_ref[...] = dis2_ref[...] * h2


def _stage_d(s3a_ref, s3b_ref, h2_ref, dis2_ref,
             w00_ref, w01_ref, b0_ref, w20_ref, w21_ref, b2_ref,
             w30_ref, w31_ref, b3_ref, wc2_ref, lw_ref, lb_ref, out_ref):
    h2 = h2_ref[...]
    tx1 = -dis2_ref[...] * (s3a_ref[...] + s3b_ref[...])
    g0 = (jnp.dot(h2, w00_ref[...], preferred_element_type=jnp.float32)
          + jnp.dot(tx1, w01_ref[...], preferred_element_type=jnp.float32)
          + b0_ref[...])
    g2 = (jnp.dot(h2, w20_ref[...], preferred_element_type=jnp.float32)
          + jnp.dot(tx1, w21_ref[...], preferred_element_type=jnp.float32)
          + b2_ref[...])
    g3 = (jnp.dot(h2, w30_ref[...], preferred_element_type=jnp.float32)
          + jnp.dot(tx1, w31_ref[...], preferred_element_type=jnp.float32)
          + b3_ref[...])
    gi = jax.nn.sigmoid(g0)
    gt = jnp.tanh(g2)
    c = gi * gt
    go = jax.nn.sigmoid(g3 + wc2_ref[...] * c)
    h = go * jnp.tanh(c)
    out_ref[...] = (jnp.dot(jnp.maximum(h, 0.0), lw_ref[...],
                            preferred_element_type=jnp.float32)
                    + lb_ref[...])


def kernel(x, edge_index, edge_attr, conv1_W, conv1_b, conv2_W, conv2_b,
           Wx, bx, Wh, bh, wc, bg, lin_W, lin_b):
    N, DIN = x.shape
    E = edge_index.shape[1]
    info = plsc.get_sparse_core_info()
    NC, NS = info.num_cores, info.num_subcores
    NW = NC * NS

    CH = -(-E // (NW * 128))
    EPAD = NW * CH * 128
    RPT = -(-(N + 1) // NS)
    RPT = -(-RPT // 128) * 128
    NPAD = RPT * NS

    f32 = jnp.float32
    src = edge_index[0].astype(jnp.int32)
    dst = edge_index[1].astype(jnp.int32)
    pad = EPAD - E
    srcs = jnp.concatenate([src, jnp.zeros((pad,), jnp.int32)]).reshape(NW, CH, 128)
    dsts = jnp.concatenate([dst, jnp.full((pad,), N, jnp.int32)]).reshape(NW, CH, 128)
    ws = jnp.concatenate([edge_attr.astype(f32), jnp.zeros((pad,), f32)]).reshape(NW, CH, 128)
    xpad = jnp.pad(x.astype(f32), ((0, NPAD - N), (0, 0)))

    D1 = conv1_W.shape[1]
    D2 = conv2_W.shape[1]

    degp = _make_deg_kernel(NC, NS, NPAD, RPT, CH)(dsts, ws)
    dga = degp[0].reshape(NPAD, 1)
    dgb = degp[1].reshape(NPAD, 1)

    xs1, dis, dis2 = _tc_call(
        _stage_a,
        (jax.ShapeDtypeStruct((NPAD, D1), f32),
         jax.ShapeDtypeStruct((NPAD, 1), f32),
         jax.ShapeDtypeStruct((NPAD, 1), f32)),
        dga, dgb, xpad, conv1_W.astype(f32))

    s1 = _make_prop_kernel(NC, NS, NPAD, RPT, CH, D1)(xs1, srcs, dsts, ws)

    (xs2,) = _tc_call(
        _stage_b,
        (jax.ShapeDtypeStruct((NPAD, D2), f32),),
        s1[0], s1[1], xs1, dis, conv2_W.astype(f32),
        conv1_b.astype(f32).reshape(1, D1))

    s2 = _make_prop_kernel(NC, NS, NPAD, RPT, CH, D2)(xs2, srcs, dsts, ws)

    h2, xs3 = _tc_call(
        _stage_c,
        (jax.ShapeDtypeStruct((NPAD, D2), f32),
         jax.ShapeDtypeStruct((NPAD, D2), f32)),
        s2[0], s2[1], xs2, dis, dis2, conv2_b.astype(f32).reshape(1, D2))

    s3 = _make_prop_kernel(NC, NS, NPAD, RPT, CH, D2)(xs3, srcs, dsts, ws)

    DH = Wx.shape[3]
    DO = lin_W.shape[1]
    bsum = (bx + bh + bg).astype(f32)
    (out,) = _tc_call(
        _stage_d,
        (jax.ShapeDtypeStruct((NPAD, DO), f32),),
        s3[0], s3[1], h2, dis2,
        Wx[0, 0].astype(f32), Wx[0, 1].astype(f32), bsum[0].reshape(1, DH),
        Wx[2, 0].astype(f32), Wx[2, 1].astype(f32), bsum[2].reshape(1, DH),
        Wx[3, 0].astype(f32), Wx[3, 1].astype(f32), bsum[3].reshape(1, DH),
        wc[2].astype(f32).reshape(1, DH),
        lin_W.astype(f32), lin_b.astype(f32).reshape(1, DO))

    return out[:N]


# speedup vs baseline: 22.8844x; 1.1004x over previous
import functools

import jax
import jax.numpy as jnp
from jax import lax
from jax.experimental import pallas as pl
from jax.experimental.pallas import tpu as pltpu
from jax.experimental.pallas import tpu_sc as plsc


def _lane_bcast(v, l):
    idx = jnp.full((16, 1), l, jnp.int32)
    return lax.gather(
        v, idx,
        lax.GatherDimensionNumbers(
            offset_dims=(), collapsed_slice_dims=(0,), start_index_map=(0,)),
        (1,), mode=lax.GatherScatterMode.PROMISE_IN_BOUNDS)

def _make_deg_kernel(NC, NS, NPAD, RPT, CH):
    mesh = plsc.VectorSubcoreMesh(core_axis_name="c", subcore_axis_name="s")

    @functools.partial(
        pl.kernel,
        out_type=jax.ShapeDtypeStruct((NC, NPAD), jnp.float32),
        mesh=mesh,
        scratch_types=[
            pltpu.VMEM((CH, 128), jnp.int32),
            pltpu.VMEM((CH, 128), jnp.float32),
            pltpu.VMEM((RPT,), jnp.float32),
            pltpu.VMEM_SHARED((NPAD,), jnp.float32),
        ],
    )
    def deg_kernel(dsts_hbm, ws_hbm, out_hbm, dst_v, w_v, io_v, acc_sh):
        cid = lax.axis_index("c")
        sid = lax.axis_index("s")
        wid = cid * NS + sid

        pltpu.sync_copy(dsts_hbm.at[wid], dst_v)
        pltpu.sync_copy(ws_hbm.at[wid], w_v)

        def zero_body(r, _):
            io_v[pl.ds(r * 16, 16)] = jnp.zeros((16,), jnp.float32)
            return 0
        lax.fori_loop(0, RPT // 16, zero_body, 0)
        pltpu.sync_copy(io_v, acc_sh.at[pl.ds(sid * RPT, RPT)])
        plsc.subcore_barrier()

        def edge_body(j, _):
            pltpu.sync_copy(w_v.at[j], acc_sh.at[dst_v.at[j]], add=True)
            return 0
        lax.fori_loop(0, CH, edge_body, 0)
        plsc.subcore_barrier()

        pltpu.sync_copy(acc_sh.at[pl.ds(sid * RPT, RPT)], io_v)
        pltpu.sync_copy(io_v, out_hbm.at[cid, pl.ds(sid * RPT, RPT)])

    return deg_kernel


def _make_prop_kernel(NC, NS, NPAD, RPT, CH, D):
    mesh = plsc.VectorSubcoreMesh(core_axis_name="c", subcore_axis_name="s")
    RB = 128
    NB = RPT // RB

    @functools.partial(
        pl.kernel,
        out_type=jax.ShapeDtypeStruct((NC, NPAD, D), jnp.float32),
        mesh=mesh,
        scratch_types=[
            pltpu.VMEM((CH, 128), jnp.int32),
            pltpu.VMEM((CH, 128), jnp.int32),
            pltpu.VMEM((CH, 128), jnp.float32),
            pltpu.VMEM((128, D), jnp.float32),
            pltpu.VMEM((128, D), jnp.float32),
            pltpu.VMEM((128, D), jnp.float32),
            pltpu.VMEM((128, D), jnp.float32),
            pltpu.VMEM((RB, D), jnp.float32),
            pltpu.VMEM_SHARED((NPAD, D), jnp.float32),
            pltpu.SemaphoreType.DMA,
            pltpu.SemaphoreType.DMA,
            pltpu.SemaphoreType.DMA,
            pltpu.SemaphoreType.DMA,
        ],
        compiler_params=pltpu.CompilerParams(use_tc_tiling_on_sc=False),
    )
    def prop_kernel(table_hbm, srcs_hbm, dsts_hbm, ws_hbm, out_hbm,
                    src_v, dst_v, w_v, g0, g1, s0, s1, io_v, acc_sh,
                    sg0, sg1, ss0, ss1):
        cid = lax.axis_index("c")
        sid = lax.axis_index("s")
        wid = cid * NS + sid

        pltpu.sync_copy(srcs_hbm.at[wid], src_v)
        pltpu.sync_copy(dsts_hbm.at[wid], dst_v)
        pltpu.sync_copy(ws_hbm.at[wid], w_v)

        def zero_body(r, _):
            for k in range(D // 16):
                io_v[r, pl.ds(k * 16, 16)] = jnp.zeros((16,), jnp.float32)
            return 0
        lax.fori_loop(0, RB, zero_body, 0)
        for c in range(NB):
            pltpu.sync_copy(io_v, acc_sh.at[pl.ds(sid * RPT + c * RB, RB)])
        plsc.subcore_barrier()

        def issue_gather(j, buf, sem):
            pltpu.async_copy(table_hbm.at[src_v.at[j]], buf, sem)

        def wait_gather(j, buf, sem):
            pltpu.make_async_copy(table_hbm.at[src_v.at[j]], buf, sem).wait()

        def issue_scatter(j, buf, sem):
            pltpu.async_copy(buf, acc_sh.at[dst_v.at[j]], sem, add=True)

        def wait_scatter(j, buf, sem):
            pltpu.make_async_copy(buf, acc_sh.at[dst_v.at[j]], sem).wait()

        def scale(j, gb, sb):
            def blk(b, _):
                w16 = w_v[j, pl.ds(b * 16, 16)]
                for l in range(16):
                    e = b * 16 + l
                    wv = _lane_bcast(w16, l)
                    for k in range(D // 16):
                        sb[e, pl.ds(k * 16, 16)] = (
                            gb[e, pl.ds(k * 16, 16)] * wv)
                return 0
            lax.fori_loop(0, 8, blk, 0)

        nhalf = CH // 2
        issue_gather(0, g0, sg0)
        issue_gather(1, g1, sg1)

        def pair_body(i, _):
            for b, gb, sb, sg, ss in ((0, g0, s0, sg0, ss0),
                                      (1, g1, s1, sg1, ss1)):
                j = 2 * i + b
                wait_gather(j, gb, sg)

                @pl.when(i >= 1)
                def _(j=j, sb=sb, ss=ss):
                    wait_scatter(j - 2, sb, ss)

                scale(j, gb, sb)
                issue_scatter(j, sb, ss)

                @pl.when(i < nhalf - 1)
                def _(j=j, gb=gb, sg=sg):
                    issue_gather(j + 2, gb, sg)
            return 0
        lax.fori_loop(0, nhalf, pair_body, 0)
        wait_scatter(CH - 2, s0, ss0)
        wait_scatter(CH - 1, s1, ss1)
        plsc.subcore_barrier()

        for c in range(NB):
            pltpu.sync_copy(acc_sh.at[pl.ds(sid * RPT + c * RB, RB)], io_v)
            pltpu.sync_copy(io_v, out_hbm.at[cid, pl.ds(sid * RPT + c * RB, RB)])

    return prop_kernel


def _tc_call(body, out_shapes, *args):
    return pl.pallas_call(
        body,
        out_shape=out_shapes,
    )(*args)


def _stage_a(dga_ref, dgb_ref, x_ref, w1_ref, xs1_ref, dis_ref, dis2_ref):
    degw = dga_ref[...] + dgb_ref[...]
    dis = lax.rsqrt(degw + 1.0)
    dis2 = jnp.where(degw > 0.0, lax.rsqrt(jnp.maximum(degw, 1e-30)), 0.0)
    xw1 = jnp.dot(x_ref[...], w1_ref[...], preferred_element_type=jnp.float32)
    xs1_ref[...] = dis * xw1
    dis_ref[...] = dis
    dis2_ref[...] = dis2


def _stage_b(s1a_ref, s1b_ref, xs1_ref, dis_ref, w2_ref, b1_ref, xs2_ref):
    h1 = jnp.maximum(
        dis_ref[...] * (s1a_ref[...] + s1b_ref[...] + xs1_ref[...])
        + b1_ref[...], 0.0)
    xs2_ref[...] = dis_ref[...] * jnp.dot(
        h1, w2_ref[...], preferred_element_type=jnp.float32)


def _stage_c(s2a_ref, s2b_ref, xs2_ref, dis_ref, dis2_ref, b2_ref,
             h2_ref, xs3_ref):
    h2 = jnp.maximum(
        dis_ref[...] * (s2a_ref[...] + s2b_ref[...] + xs2_ref[...])
        + b2_ref[...], 0.0)
    h2_ref[...] = h2
    xs3_ref[...] = dis2_ref[...] * h2


def _stage_d(s3a_ref, s3b_ref, h2_ref, dis2_ref,
             w00_ref, w01_ref, b0_ref, w20_ref, w21_ref, b2_ref,
             w30_ref, w31_ref, b3_ref, wc2_ref, lw_ref, lb_ref, out_ref):
    h2 = h2_ref[...]
    tx1 = -dis2_ref[...] * (s3a_ref[...] + s3b_ref[...])
    g0 = (jnp.dot(h2, w00_ref[...], preferred_element_type=jnp.float32)
          + jnp.dot(tx1, w01_ref[...], preferred_element_type=jnp.float32)
          + b0_ref[...])
    g2 = (jnp.dot(h2, w20_ref[...], preferred_element_type=jnp.float32)
          + jnp.dot(tx1, w21_ref[...], preferred_element_type=jnp.float32)
          + b2_ref[...])
    g3 = (jnp.dot(h2, w30_ref[...], preferred_element_type=jnp.float32)
          + jnp.dot(tx1, w31_ref[...], preferred_element_type=jnp.float32)
          + b3_ref[...])
    gi = jax.nn.sigmoid(g0)
    gt = jnp.tanh(g2)
    c = gi * gt
    go = jax.nn.sigmoid(g3 + wc2_ref[...] * c)
    h = go * jnp.tanh(c)
    out_ref[...] = (jnp.dot(jnp.maximum(h, 0.0), lw_ref[...],
                            preferred_element_type=jnp.float32)
                    + lb_ref[...])


def kernel(x, edge_index, edge_attr, conv1_W, conv1_b, conv2_W, conv2_b,
           Wx, bx, Wh, bh, wc, bg, lin_W, lin_b):
    N, DIN = x.shape
    E = edge_index.shape[1]
    info = plsc.get_sparse_core_info()
    NC, NS = info.num_cores, info.num_subcores
    NW = NC * NS

    CH = -(-E // (NW * 128))
    CH = CH + (CH % 2)
    EPAD = NW * CH * 128
    RPT = -(-(N + 1) // NS)
    RPT = -(-RPT // 128) * 128
    NPAD = RPT * NS

    f32 = jnp.float32
    src = edge_index[0].astype(jnp.int32)
    dst = edge_index[1].astype(jnp.int32)
    pad = EPAD - E
    srcs = jnp.concatenate([src, jnp.zeros((pad,), jnp.int32)]).reshape(NW, CH, 128)
    dsts = jnp.concatenate([dst, jnp.full((pad,), N, jnp.int32)]).reshape(NW, CH, 128)
    ws = jnp.concatenate([edge_attr.astype(f32), jnp.zeros((pad,), f32)]).reshape(NW, CH, 128)
    xpad = jnp.pad(x.astype(f32), ((0, NPAD - N), (0, 0)))

    D1 = conv1_W.shape[1]
    D2 = conv2_W.shape[1]

    degp = _make_deg_kernel(NC, NS, NPAD, RPT, CH)(dsts, ws)
    dga = degp[0].reshape(NPAD, 1)
    dgb = degp[1].reshape(NPAD, 1)

    xs1, dis, dis2 = _tc_call(
        _stage_a,
        (jax.ShapeDtypeStruct((NPAD, D1), f32),
         jax.ShapeDtypeStruct((NPAD, 1), f32),
         jax.ShapeDtypeStruct((NPAD, 1), f32)),
        dga, dgb, xpad, conv1_W.astype(f32))

    s1 = _make_prop_kernel(NC, NS, NPAD, RPT, CH, D1)(xs1, srcs, dsts, ws)

    (xs2,) = _tc_call(
        _stage_b,
        (jax.ShapeDtypeStruct((NPAD, D2), f32),),
        s1[0], s1[1], xs1, dis, conv2_W.astype(f32),
        conv1_b.astype(f32).reshape(1, D1))

    s2 = _make_prop_kernel(NC, NS, NPAD, RPT, CH, D2)(xs2, srcs, dsts, ws)

    h2, xs3 = _tc_call(
        _stage_c,
        (jax.ShapeDtypeStruct((NPAD, D2), f32),
         jax.ShapeDtypeStruct((NPAD, D2), f32)),
        s2[0], s2[1], xs2, dis, dis2, conv2_b.astype(f32).reshape(1, D2))

    s3 = _make_prop_kernel(NC, NS, NPAD, RPT, CH, D2)(xs3, srcs, dsts, ws)

    DH = Wx.shape[3]
    DO = lin_W.shape[1]
    bsum = (bx + bh + bg).astype(f32)
    (out,) = _tc_call(
        _stage_d,
        (jax.ShapeDtypeStruct((NPAD, DO), f32),),
        s3[0], s3[1], h2, dis2,
        Wx[0, 0].astype(f32), Wx[0, 1].astype(f32), bsum[0].reshape(1, DH),
        Wx[2, 0].astype(f32), Wx[2, 1].astype(f32), bsum[2].reshape(1, DH),
        Wx[3, 0].astype(f32), Wx[3, 1].astype(f32), bsum[3].reshape(1, DH),
        wc[2].astype(f32).reshape(1, DH),
        lin_W.astype(f32), lin_b.astype(f32).reshape(1, DO))

    return out[:N]


# speedup vs baseline: 39.0567x; 1.7067x over previous
import functools

import jax
import jax.numpy as jnp
from jax import lax
from jax.experimental import pallas as pl
from jax.experimental.pallas import tpu as pltpu
from jax.experimental.pallas import tpu_sc as plsc


def _lane_bcast(v, l):
    idx = jnp.full((16, 1), l, jnp.int32)
    return lax.gather(
        v, idx,
        lax.GatherDimensionNumbers(
            offset_dims=(), collapsed_slice_dims=(0,), start_index_map=(0,)),
        (1,), mode=lax.GatherScatterMode.PROMISE_IN_BOUNDS)

def _make_deg_kernel(NC, NS, NPAD, RPT, CH):
    mesh = plsc.VectorSubcoreMesh(core_axis_name="c", subcore_axis_name="s")

    @functools.partial(
        pl.kernel,
        out_type=jax.ShapeDtypeStruct((NC, NPAD), jnp.float32),
        mesh=mesh,
        scratch_types=[
            pltpu.VMEM((CH, 128), jnp.int32),
            pltpu.VMEM((CH, 128), jnp.float32),
            pltpu.VMEM((RPT,), jnp.float32),
            pltpu.VMEM_SHARED((NPAD,), jnp.float32),
        ],
    )
    def deg_kernel(dsts_hbm, ws_hbm, out_hbm, dst_v, w_v, io_v, acc_sh):
        cid = lax.axis_index("c")
        sid = lax.axis_index("s")
        wid = cid * NS + sid

        pltpu.sync_copy(dsts_hbm.at[wid], dst_v)
        pltpu.sync_copy(ws_hbm.at[wid], w_v)

        def zero_body(r, _):
            io_v[pl.ds(r * 16, 16)] = jnp.zeros((16,), jnp.float32)
            return 0
        lax.fori_loop(0, RPT // 16, zero_body, 0)
        pltpu.sync_copy(io_v, acc_sh.at[pl.ds(sid * RPT, RPT)])
        plsc.subcore_barrier()

        def edge_body(j, _):
            pltpu.sync_copy(w_v.at[j], acc_sh.at[dst_v.at[j]], add=True)
            return 0
        lax.fori_loop(0, CH, edge_body, 0)
        plsc.subcore_barrier()

        pltpu.sync_copy(acc_sh.at[pl.ds(sid * RPT, RPT)], io_v)
        pltpu.sync_copy(io_v, out_hbm.at[cid, pl.ds(sid * RPT, RPT)])

    return deg_kernel


def _make_prop_kernel(NC, NS, NPAD, RPT, CH, D):
    mesh = plsc.VectorSubcoreMesh(core_axis_name="c", subcore_axis_name="s")
    RB = 128
    NB = RPT // RB

    @functools.partial(
        pl.kernel,
        out_type=jax.ShapeDtypeStruct((NC, NPAD, D), jnp.float32),
        mesh=mesh,
        scratch_types=[
            pltpu.VMEM((CH, 128), jnp.int32),
            pltpu.VMEM((CH, 128), jnp.int32),
            pltpu.VMEM((CH, 128), jnp.float32),
            pltpu.VMEM((128, D), jnp.float32),
            pltpu.VMEM((128, D), jnp.float32),
            pltpu.VMEM((128, D), jnp.float32),
            pltpu.VMEM((128, D), jnp.float32),
            pltpu.VMEM((RB, D), jnp.float32),
            pltpu.VMEM_SHARED((NPAD, D), jnp.float32),
            pltpu.VMEM_SHARED((NPAD, D), jnp.float32),
            pltpu.SemaphoreType.DMA,
            pltpu.SemaphoreType.DMA,
            pltpu.SemaphoreType.DMA,
            pltpu.SemaphoreType.DMA,
        ],
        compiler_params=pltpu.CompilerParams(use_tc_tiling_on_sc=False),
    )
    def prop_kernel(table_hbm, srcs_hbm, dsts_hbm, ws_hbm, out_hbm,
                    src_v, dst_v, w_v, g0, g1, s0, s1, io_v, acc_sh,
                    tab_sh, sg0, sg1, ss0, ss1):
        cid = lax.axis_index("c")
        sid = lax.axis_index("s")
        wid = cid * NS + sid

        pltpu.sync_copy(srcs_hbm.at[wid], src_v)
        pltpu.sync_copy(dsts_hbm.at[wid], dst_v)
        pltpu.sync_copy(ws_hbm.at[wid], w_v)

        def zero_body(r, _):
            for k in range(D // 16):
                io_v[r, pl.ds(k * 16, 16)] = jnp.zeros((16,), jnp.float32)
            return 0
        lax.fori_loop(0, RB, zero_body, 0)
        for c in range(NB):
            pltpu.sync_copy(io_v, acc_sh.at[pl.ds(sid * RPT + c * RB, RB)])
        for c in range(NB):
            pltpu.sync_copy(table_hbm.at[pl.ds(sid * RPT + c * RB, RB)],
                            tab_sh.at[pl.ds(sid * RPT + c * RB, RB)])
        plsc.subcore_barrier()

        def issue_gather(j, buf, sem):
            pltpu.async_copy(tab_sh.at[src_v.at[j]], buf, sem)

        def wait_gather(j, buf, sem):
            pltpu.make_async_copy(tab_sh.at[src_v.at[j]], buf, sem).wait()

        def issue_scatter(j, buf, sem):
            pltpu.async_copy(buf, acc_sh.at[dst_v.at[j]], sem, add=True)

        def wait_scatter(j, buf, sem):
            pltpu.make_async_copy(buf, acc_sh.at[dst_v.at[j]], sem).wait()

        def scale(j, gb, sb):
            def blk(b, _):
                w16 = w_v[j, pl.ds(b * 16, 16)]
                for l in range(16):
                    e = b * 16 + l
                    wv = _lane_bcast(w16, l)
                    for k in range(D // 16):
                        sb[e, pl.ds(k * 16, 16)] = (
                            gb[e, pl.ds(k * 16, 16)] * wv)
                return 0
            lax.fori_loop(0, 8, blk, 0)

        nhalf = CH // 2
        issue_gather(0, g0, sg0)
        issue_gather(1, g1, sg1)

        def pair_body(i, _):
            for b, gb, sb, sg, ss in ((0, g0, s0, sg0, ss0),
                                      (1, g1, s1, sg1, ss1)):
                j = 2 * i + b
                wait_gather(j, gb, sg)

                @pl.when(i >= 1)
                def _(j=j, sb=sb, ss=ss):
                    wait_scatter(j - 2, sb, ss)

                scale(j, gb, sb)
                issue_scatter(j, sb, ss)

                @pl.when(i < nhalf - 1)
                def _(j=j, gb=gb, sg=sg):
                    issue_gather(j + 2, gb, sg)
            return 0
        lax.fori_loop(0, nhalf, pair_body, 0)
        wait_scatter(CH - 2, s0, ss0)
        wait_scatter(CH - 1, s1, ss1)
        plsc.subcore_barrier()

        for c in range(NB):
            pltpu.sync_copy(acc_sh.at[pl.ds(sid * RPT + c * RB, RB)], io_v)
            pltpu.sync_copy(io_v, out_hbm.at[cid, pl.ds(sid * RPT + c * RB, RB)])

    return prop_kernel


def _tc_call(body, out_shapes, *args):
    return pl.pallas_call(
        body,
        out_shape=out_shapes,
    )(*args)


def _stage_a(dga_ref, dgb_ref, x_ref, w1_ref, xs1_ref, dis_ref, dis2_ref):
    degw = dga_ref[...] + dgb_ref[...]
    dis = lax.rsqrt(degw + 1.0)
    dis2 = jnp.where(degw > 0.0, lax.rsqrt(jnp.maximum(degw, 1e-30)), 0.0)
    xw1 = jnp.dot(x_ref[...], w1_ref[...], preferred_element_type=jnp.float32)
    xs1_ref[...] = dis * xw1
    dis_ref[...] = dis
    dis2_ref[...] = dis2


def _stage_b(sa0_ref, sa1_ref, sb0_ref, sb1_ref, xs1a_ref, xs1b_ref,
             dis_ref, w2a_ref, w2b_ref, b1a_ref, b1b_ref, xs2_ref):
    h1a = jnp.maximum(
        dis_ref[...] * (sa0_ref[...] + sa1_ref[...] + xs1a_ref[...])
        + b1a_ref[...], 0.0)
    h1b = jnp.maximum(
        dis_ref[...] * (sb0_ref[...] + sb1_ref[...] + xs1b_ref[...])
        + b1b_ref[...], 0.0)
    xs2_ref[...] = dis_ref[...] * (
        jnp.dot(h1a, w2a_ref[...], preferred_element_type=jnp.float32)
        + jnp.dot(h1b, w2b_ref[...], preferred_element_type=jnp.float32))


def _stage_c(s2a_ref, s2b_ref, xs2_ref, dis_ref, dis2_ref, b2_ref,
             h2_ref, xs3_ref):
    h2 = jnp.maximum(
        dis_ref[...] * (s2a_ref[...] + s2b_ref[...] + xs2_ref[...])
        + b2_ref[...], 0.0)
    h2_ref[...] = h2
    xs3_ref[...] = dis2_ref[...] * h2


def _stage_d(s3a_ref, s3b_ref, h2_ref, dis2_ref,
             w00_ref, w01_ref, b0_ref, w20_ref, w21_ref, b2_ref,
             w30_ref, w31_ref, b3_ref, wc2_ref, lw_ref, lb_ref, out_ref):
    h2 = h2_ref[...]
    tx1 = -dis2_ref[...] * (s3a_ref[...] + s3b_ref[...])
    g0 = (jnp.dot(h2, w00_ref[...], preferred_element_type=jnp.float32)
          + jnp.dot(tx1, w01_ref[...], preferred_element_type=jnp.float32)
          + b0_ref[...])
    g2 = (jnp.dot(h2, w20_ref[...], preferred_element_type=jnp.float32)
          + jnp.dot(tx1, w21_ref[...], preferred_element_type=jnp.float32)
          + b2_ref[...])
    g3 = (jnp.dot(h2, w30_ref[...], preferred_element_type=jnp.float32)
          + jnp.dot(tx1, w31_ref[...], preferred_element_type=jnp.float32)
          + b3_ref[...])
    gi = jax.nn.sigmoid(g0)
    gt = jnp.tanh(g2)
    c = gi * gt
    go = jax.nn.sigmoid(g3 + wc2_ref[...] * c)
    h = go * jnp.tanh(c)
    out_ref[...] = (jnp.dot(jnp.maximum(h, 0.0), lw_ref[...],
                            preferred_element_type=jnp.float32)
                    + lb_ref[...])


def kernel(x, edge_index, edge_attr, conv1_W, conv1_b, conv2_W, conv2_b,
           Wx, bx, Wh, bh, wc, bg, lin_W, lin_b):
    N, DIN = x.shape
    E = edge_index.shape[1]
    info = plsc.get_sparse_core_info()
    NC, NS = info.num_cores, info.num_subcores
    NW = NC * NS

    CH = -(-E // (NW * 128))
    CH = CH + (CH % 2)
    EPAD = NW * CH * 128
    RPT = -(-(N + 1) // NS)
    RPT = -(-RPT // 128) * 128
    NPAD = RPT * NS

    f32 = jnp.float32
    src = edge_index[0].astype(jnp.int32)
    dst = edge_index[1].astype(jnp.int32)
    pad = EPAD - E
    srcs = jnp.concatenate([src, jnp.zeros((pad,), jnp.int32)]).reshape(NW, CH, 128)
    dsts = jnp.concatenate([dst, jnp.full((pad,), N, jnp.int32)]).reshape(NW, CH, 128)
    ws = jnp.concatenate([edge_attr.astype(f32), jnp.zeros((pad,), f32)]).reshape(NW, CH, 128)
    xpad = jnp.pad(x.astype(f32), ((0, NPAD - N), (0, 0)))

    D1 = conv1_W.shape[1]
    D2 = conv2_W.shape[1]

    degp = _make_deg_kernel(NC, NS, NPAD, RPT, CH)(dsts, ws)
    dga = degp[0].reshape(NPAD, 1)
    dgb = degp[1].reshape(NPAD, 1)

    xs1, dis, dis2 = _tc_call(
        _stage_a,
        (jax.ShapeDtypeStruct((NPAD, D1), f32),
         jax.ShapeDtypeStruct((NPAD, 1), f32),
         jax.ShapeDtypeStruct((NPAD, 1), f32)),
        dga, dgb, xpad, conv1_W.astype(f32))

    DH1 = D1 // 2
    prop32 = _make_prop_kernel(NC, NS, NPAD, RPT, CH, DH1)
    xs1a = xs1[:, :DH1]
    xs1b = xs1[:, DH1:]
    s1a = prop32(xs1a, srcs, dsts, ws)
    s1b = prop32(xs1b, srcs, dsts, ws)

    w2 = conv2_W.astype(f32)
    b1 = conv1_b.astype(f32)
    (xs2,) = _tc_call(
        _stage_b,
        (jax.ShapeDtypeStruct((NPAD, D2), f32),),
        s1a[0], s1a[1], s1b[0], s1b[1], xs1a, xs1b, dis,
        w2[:DH1], w2[DH1:],
        b1[:DH1].reshape(1, DH1), b1[DH1:].reshape(1, DH1))

    s2 = _make_prop_kernel(NC, NS, NPAD, RPT, CH, D2)(xs2, srcs, dsts, ws)

    h2, xs3 = _tc_call(
        _stage_c,
        (jax.ShapeDtypeStruct((NPAD, D2), f32),
         jax.ShapeDtypeStruct((NPAD, D2), f32)),
        s2[0], s2[1], xs2, dis, dis2, conv2_b.astype(f32).reshape(1, D2))

    s3 = _make_prop_kernel(NC, NS, NPAD, RPT, CH, D2)(xs3, srcs, dsts, ws)

    DH = Wx.shape[3]
    DO = lin_W.shape[1]
    bsum = (bx + bh + bg).astype(f32)
    (out,) = _tc_call(
        _stage_d,
        (jax.ShapeDtypeStruct((NPAD, DO), f32),),
        s3[0], s3[1], h2, dis2,
        Wx[0, 0].astype(f32), Wx[0, 1].astype(f32), bsum[0].reshape(1, DH),
        Wx[2, 0].astype(f32), Wx[2, 1].astype(f32), bsum[2].reshape(1, DH),
        Wx[3, 0].astype(f32), Wx[3, 1].astype(f32), bsum[3].reshape(1, DH),
        wc[2].astype(f32).reshape(1, DH),
        lin_W.astype(f32), lin_b.astype(f32).reshape(1, DO))

    return out[:N]


# speedup vs baseline: 43.5994x; 1.1163x over previous
import functools

import jax
import jax.numpy as jnp
from jax import lax
from jax.experimental import pallas as pl
from jax.experimental.pallas import tpu as pltpu
from jax.experimental.pallas import tpu_sc as plsc


def _lane_bcast(v, l):
    idx = jnp.full((16, 1), l, jnp.int32)
    return lax.gather(
        v, idx,
        lax.GatherDimensionNumbers(
            offset_dims=(), collapsed_slice_dims=(0,), start_index_map=(0,)),
        (1,), mode=lax.GatherScatterMode.PROMISE_IN_BOUNDS)

def _make_deg_kernel(NC, NS, NPAD, RPT, CH):
    mesh = plsc.VectorSubcoreMesh(core_axis_name="c", subcore_axis_name="s")

    @functools.partial(
        pl.kernel,
        out_type=jax.ShapeDtypeStruct((NC, NPAD), jnp.float32),
        mesh=mesh,
        scratch_types=[
            pltpu.VMEM((CH, 128), jnp.int32),
            pltpu.VMEM((CH, 128), jnp.float32),
            pltpu.VMEM((RPT,), jnp.float32),
            pltpu.VMEM_SHARED((NPAD,), jnp.float32),
        ],
    )
    def deg_kernel(dsts_hbm, ws_hbm, out_hbm, dst_v, w_v, io_v, acc_sh):
        cid = lax.axis_index("c")
        sid = lax.axis_index("s")
        wid = cid * NS + sid

        pltpu.sync_copy(dsts_hbm.at[wid], dst_v)
        pltpu.sync_copy(ws_hbm.at[wid], w_v)

        def zero_body(r, _):
            io_v[pl.ds(r * 16, 16)] = jnp.zeros((16,), jnp.float32)
            return 0
        lax.fori_loop(0, RPT // 16, zero_body, 0)
        pltpu.sync_copy(io_v, acc_sh.at[pl.ds(sid * RPT, RPT)])
        plsc.subcore_barrier()

        def edge_body(j, _):
            pltpu.sync_copy(w_v.at[j], acc_sh.at[dst_v.at[j]], add=True)
            return 0
        lax.fori_loop(0, CH, edge_body, 0)
        plsc.subcore_barrier()

        pltpu.sync_copy(acc_sh.at[pl.ds(sid * RPT, RPT)], io_v)
        pltpu.sync_copy(io_v, out_hbm.at[cid, pl.ds(sid * RPT, RPT)])

    return deg_kernel


def _make_prop_kernel(NC, NS, NPAD, RPT, CH, D):
    mesh = plsc.VectorSubcoreMesh(core_axis_name="c", subcore_axis_name="s")
    RB = 128
    NB = RPT // RB

    @functools.partial(
        pl.kernel,
        out_type=jax.ShapeDtypeStruct((NC, NPAD, D), jnp.float32),
        mesh=mesh,
        scratch_types=[
            pltpu.VMEM((CH, 128), jnp.int32),
            pltpu.VMEM((CH, 128), jnp.int32),
            pltpu.VMEM((CH, 128), jnp.float32),
            pltpu.VMEM((128, D), jnp.float32),
            pltpu.VMEM((128, D), jnp.float32),
            pltpu.VMEM((128, D), jnp.float32),
            pltpu.VMEM((128, D), jnp.float32),
            pltpu.VMEM((RB, D), jnp.float32),
            pltpu.VMEM_SHARED((NPAD, D), jnp.float32),
            pltpu.VMEM_SHARED((NPAD, D), jnp.float32),
            pltpu.SemaphoreType.DMA,
            pltpu.SemaphoreType.DMA,
            pltpu.SemaphoreType.DMA,
            pltpu.SemaphoreType.DMA,
        ],
        compiler_params=pltpu.CompilerParams(use_tc_tiling_on_sc=False),
    )
    def prop_kernel(table_hbm, srcs_hbm, dsts_hbm, ws_hbm, out_hbm,
                    src_v, dst_v, w_v, g0, g1, s0, s1, io_v, acc_sh,
                    tab_sh, sg0, sg1, ss0, ss1):
        cid = lax.axis_index("c")
        sid = lax.axis_index("s")
        wid = cid * NS + sid

        pltpu.sync_copy(srcs_hbm.at[wid], src_v)
        pltpu.sync_copy(dsts_hbm.at[wid], dst_v)
        pltpu.sync_copy(ws_hbm.at[wid], w_v)

        def zero_body(r, _):
            for k in range(D // 16):
                io_v[r, pl.ds(k * 16, 16)] = jnp.zeros((16,), jnp.float32)
            return 0
        lax.fori_loop(0, RB, zero_body, 0)
        for c in range(NB):
            pltpu.sync_copy(io_v, acc_sh.at[pl.ds(sid * RPT + c * RB, RB)])
        for c in range(NB):
            pltpu.sync_copy(table_hbm.at[pl.ds(sid * RPT + c * RB, RB)],
                            tab_sh.at[pl.ds(sid * RPT + c * RB, RB)])
        plsc.subcore_barrier()

        def issue_gather(j, buf, sem):
            pltpu.async_copy(tab_sh.at[src_v.at[j]], buf, sem)

        def wait_gather(j, buf, sem):
            pltpu.make_async_copy(tab_sh.at[src_v.at[j]], buf, sem).wait()

        def issue_scatter(j, buf, sem):
            pltpu.async_copy(buf, acc_sh.at[dst_v.at[j]], sem, add=True)

        def wait_scatter(j, buf, sem):
            pltpu.make_async_copy(buf, acc_sh.at[dst_v.at[j]], sem).wait()

        def scale(j, gb, sb):
            def blk(b, _):
                w16 = w_v[j, pl.ds(b * 16, 16)]
                for l in range(16):
                    e = b * 16 + l
                    wv = _lane_bcast(w16, l)
                    for k in range(D // 16):
                        sb[e, pl.ds(k * 16, 16)] = (
                            gb[e, pl.ds(k * 16, 16)] * wv)
                return 0
            lax.fori_loop(0, 8, blk, 0)

        nhalf = CH // 2
        issue_gather(0, g0, sg0)
        issue_gather(1, g1, sg1)

        def pair_body(i, _):
            for b, gb, sb, sg, ss in ((0, g0, s0, sg0, ss0),
                                      (1, g1, s1, sg1, ss1)):
                j = 2 * i + b
                wait_gather(j, gb, sg)

                @pl.when(i >= 1)
                def _(j=j, sb=sb, ss=ss):
                    wait_scatter(j - 2, sb, ss)

                scale(j, gb, sb)
                issue_scatter(j, sb, ss)

                @pl.when(i < nhalf - 1)
                def _(j=j, gb=gb, sg=sg):
                    issue_gather(j + 2, gb, sg)
            return 0
        lax.fori_loop(0, nhalf, pair_body, 0)
        wait_scatter(CH - 2, s0, ss0)
        wait_scatter(CH - 1, s1, ss1)
        plsc.subcore_barrier()

        for c in range(NB):
            pltpu.sync_copy(acc_sh.at[pl.ds(sid * RPT + c * RB, RB)], io_v)
            pltpu.sync_copy(io_v, out_hbm.at[cid, pl.ds(sid * RPT + c * RB, RB)])

    return prop_kernel


def _tc_call(body, out_shapes, *args):
    return pl.pallas_call(
        body,
        out_shape=out_shapes,
    )(*args)


def _stage_a(dg0_ref, dg1_ref, x4_ref, w1a_ref, w1b_ref,
             xs1a_ref, xs1b_ref, dise_ref, dis2e_ref):
    degw = dg0_ref[...] + dg1_ref[...]
    dis = lax.rsqrt(degw + 1.0)
    dis2 = jnp.where(degw > 0.0, lax.rsqrt(jnp.maximum(degw, 1e-30)), 0.0)
    x4 = x4_ref[...]
    xs1a_ref[...] = dis * jnp.dot(x4, w1a_ref[...],
                                  preferred_element_type=jnp.float32)
    xs1b_ref[...] = dis * jnp.dot(x4, w1b_ref[...],
                                  preferred_element_type=jnp.float32)
    dise_ref[...] = dis
    dis2e_ref[...] = dis2


def _stage_b(sa0_ref, sa1_ref, sb0_ref, sb1_ref, xs1a_ref, xs1b_ref,
             dise_ref, w2a_ref, w2b_ref, b1a_ref, b1b_ref, xs2_ref):
    dise = dise_ref[...]
    h1a = jnp.maximum(
        dise * (sa0_ref[...] + sa1_ref[...] + xs1a_ref[...])
        + b1a_ref[...], 0.0)
    h1b = jnp.maximum(
        dise * (sb0_ref[...] + sb1_ref[...] + xs1b_ref[...])
        + b1b_ref[...], 0.0)
    xs2_ref[...] = dise * (
        jnp.dot(h1a, w2a_ref[...], preferred_element_type=jnp.float32)
        + jnp.dot(h1b, w2b_ref[...], preferred_element_type=jnp.float32))


def _stage_c(s2a_ref, s2b_ref, xs2_ref, dise_ref, dis2e_ref, b2_ref,
             h2_ref, xs3_ref):
    h2 = jnp.maximum(
        dise_ref[...] * (s2a_ref[...] + s2b_ref[...] + xs2_ref[...])
        + b2_ref[...], 0.0)
    h2_ref[...] = h2
    xs3_ref[...] = dis2e_ref[...] * h2


def _stage_d(s3a_ref, s3b_ref, h2_ref, dis2e_ref,
             w00_ref, w01_ref, b0_ref, w20_ref, w21_ref, b2_ref,
             w30_ref, w31_ref, b3_ref, wc2_ref, lw_ref, lb_ref, out_ref):
    h2 = h2_ref[...]
    tx1 = -dis2e_ref[...] * (s3a_ref[...] + s3b_ref[...])
    g0 = (jnp.dot(h2, w00_ref[...], preferred_element_type=jnp.float32)
          + jnp.dot(tx1, w01_ref[...], preferred_element_type=jnp.float32)
          + b0_ref[...])
    g2 = (jnp.dot(h2, w20_ref[...], preferred_element_type=jnp.float32)
          + jnp.dot(tx1, w21_ref[...], preferred_element_type=jnp.float32)
          + b2_ref[...])
    g3 = (jnp.dot(h2, w30_ref[...], preferred_element_type=jnp.float32)
          + jnp.dot(tx1, w31_ref[...], preferred_element_type=jnp.float32)
          + b3_ref[...])
    gi = jax.nn.sigmoid(g0)
    gt = jnp.tanh(g2)
    c = gi * gt
    go = jax.nn.sigmoid(g3 + wc2_ref[...] * c)
    h = go * jnp.tanh(c)
    out_ref[...] = (jnp.dot(jnp.maximum(h, 0.0), lw_ref[...],
                            preferred_element_type=jnp.float32)
                    + lb_ref[...])


def kernel(x, edge_index, edge_attr, conv1_W, conv1_b, conv2_W, conv2_b,
           Wx, bx, Wh, bh, wc, bg, lin_W, lin_b):
    N, DIN = x.shape
    E = edge_index.shape[1]
    info = plsc.get_sparse_core_info()
    NC, NS = info.num_cores, info.num_subcores
    NW = NC * NS

    CH = -(-E // (NW * 128))
    CH = CH + (CH % 2)
    EPAD = NW * CH * 128
    RPT = -(-(N + 1) // NS)
    RPT = -(-RPT // 128) * 128
    NPAD = RPT * NS

    f32 = jnp.float32
    src = edge_index[0].astype(jnp.int32)
    dst = edge_index[1].astype(jnp.int32)
    pad = EPAD - E
    srcs = jnp.concatenate([src, jnp.zeros((pad,), jnp.int32)]).reshape(NW, CH, 128)
    dsts = jnp.concatenate([dst, jnp.full((pad,), N, jnp.int32)]).reshape(NW, CH, 128)
    ws = jnp.concatenate([edge_attr.astype(f32), jnp.zeros((pad,), f32)]).reshape(NW, CH, 128)
    xpad = jnp.pad(x.astype(f32), ((0, NPAD - N), (0, 0)))

    D1 = conv1_W.shape[1]
    D2 = conv2_W.shape[1]
    DH1 = D1 // 2
    PK = NPAD // 4

    def blk4(w):
        return jnp.kron(jnp.eye(4, dtype=f32), w.astype(f32))

    def tile4(b):
        return jnp.tile(b.astype(f32), 4).reshape(1, -1)

    pk = jax.ShapeDtypeStruct((PK, 128), f32)

    degp = _make_deg_kernel(NC, NS, NPAD, RPT, CH)(dsts, ws)
    dg0 = jnp.broadcast_to(degp[0].reshape(PK, 4, 1), (PK, 4, 32)).reshape(PK, 128)
    dg1 = jnp.broadcast_to(degp[1].reshape(PK, 4, 1), (PK, 4, 32)).reshape(PK, 128)

    w1 = conv1_W.astype(f32)
    xs1a_pk, xs1b_pk, dise, dis2e = _tc_call(
        _stage_a, (pk, pk, pk, pk),
        dg0, dg1, xpad.reshape(PK, 4 * DIN),
        blk4(w1[:, :DH1]), blk4(w1[:, DH1:]))

    prop32 = _make_prop_kernel(NC, NS, NPAD, RPT, CH, DH1)
    s1a = prop32(xs1a_pk.reshape(NPAD, DH1), srcs, dsts, ws)
    s1b = prop32(xs1b_pk.reshape(NPAD, DH1), srcs, dsts, ws)

    def pk2(part):
        return part[0].reshape(PK, 128), part[1].reshape(PK, 128)

    w2 = conv2_W.astype(f32)
    b1 = conv1_b.astype(f32)
    (xs2_pk,) = _tc_call(
        _stage_b, (pk,),
        *pk2(s1a), *pk2(s1b), xs1a_pk, xs1b_pk, dise,
        blk4(w2[:DH1]), blk4(w2[DH1:]),
        tile4(b1[:DH1]), tile4(b1[DH1:]))

    s2 = prop32(xs2_pk.reshape(NPAD, D2), srcs, dsts, ws)

    h2_pk, xs3_pk = _tc_call(
        _stage_c, (pk, pk),
        *pk2(s2), xs2_pk, dise, dis2e, tile4(conv2_b))

    s3 = prop32(xs3_pk.reshape(NPAD, D2), srcs, dsts, ws)

    DG = Wx.shape[3]
    DO = lin_W.shape[1]
    bsum = (bx + bh + bg).astype(f32)
    (out_pk,) = _tc_call(
        _stage_d,
        (jax.ShapeDtypeStruct((PK, 4 * DO), f32),),
        *pk2(s3), h2_pk, dis2e,
        blk4(Wx[0, 0]), blk4(Wx[0, 1]), tile4(bsum[0]),
        blk4(Wx[2, 0]), blk4(Wx[2, 1]), tile4(bsum[2]),
        blk4(Wx[3, 0]), blk4(Wx[3, 1]), tile4(bsum[3]),
        tile4(wc[2]), blk4(lin_W), tile4(lin_b))

    return out_pk.reshape(NPAD, DO)[:N]


# speedup vs baseline: 53.6014x; 1.2294x over previous
import functools

import jax
import jax.numpy as jnp
from jax import lax
from jax.experimental import pallas as pl
from jax.experimental.pallas import tpu as pltpu
from jax.experimental.pallas import tpu_sc as plsc


def _lane_bcast(v, l):
    idx = jnp.full((16, 1), l, jnp.int32)
    return lax.gather(
        v, idx,
        lax.GatherDimensionNumbers(
            offset_dims=(), collapsed_slice_dims=(0,), start_index_map=(0,)),
        (1,), mode=lax.GatherScatterMode.PROMISE_IN_BOUNDS)

def _make_deg_kernel(NC, NS, NPAD, RPT, CH):
    mesh = plsc.VectorSubcoreMesh(core_axis_name="c", subcore_axis_name="s")

    @functools.partial(
        pl.kernel,
        out_type=jax.ShapeDtypeStruct((NC, NPAD), jnp.float32),
        mesh=mesh,
        scratch_types=[
            pltpu.VMEM((CH, 128), jnp.int32),
            pltpu.VMEM((CH, 128), jnp.float32),
            pltpu.VMEM((RPT,), jnp.float32),
            pltpu.VMEM_SHARED((NPAD,), jnp.float32),
        ],
    )
    def deg_kernel(dsts_hbm, ws_hbm, out_hbm, dst_v, w_v, io_v, acc_sh):
        cid = lax.axis_index("c")
        sid = lax.axis_index("s")
        wid = cid * NS + sid

        pltpu.sync_copy(dsts_hbm.at[wid], dst_v)
        pltpu.sync_copy(ws_hbm.at[wid], w_v)

        def zero_body(r, _):
            io_v[pl.ds(r * 16, 16)] = jnp.zeros((16,), jnp.float32)
            return 0
        lax.fori_loop(0, RPT // 16, zero_body, 0)
        pltpu.sync_copy(io_v, acc_sh.at[pl.ds(sid * RPT, RPT)])
        plsc.subcore_barrier()

        def edge_body(j, _):
            pltpu.sync_copy(w_v.at[j], acc_sh.at[dst_v.at[j]], add=True)
            return 0
        lax.fori_loop(0, CH, edge_body, 0)
        plsc.subcore_barrier()

        pltpu.sync_copy(acc_sh.at[pl.ds(sid * RPT, RPT)], io_v)
        pltpu.sync_copy(io_v, out_hbm.at[cid, pl.ds(sid * RPT, RPT)])

    return deg_kernel


def _make_prop_kernel(NC, NS, NPAD, RPT, CH, D):
    mesh = plsc.VectorSubcoreMesh(core_axis_name="c", subcore_axis_name="s")
    RB = 128
    NB = RPT // RB

    @functools.partial(
        pl.kernel,
        out_type=jax.ShapeDtypeStruct((NC, NPAD, D), jnp.float32),
        mesh=mesh,
        scratch_types=[
            pltpu.VMEM((CH, 128), jnp.int32),
            pltpu.VMEM((CH, 128), jnp.int32),
            pltpu.VMEM((CH, 128), jnp.float32),
            pltpu.VMEM((128, D), jnp.float32),
            pltpu.VMEM((128, D), jnp.float32),
            pltpu.VMEM((128, D), jnp.float32),
            pltpu.VMEM((128, D), jnp.float32),
            pltpu.VMEM((RB, D), jnp.float32),
            pltpu.VMEM_SHARED((NPAD, D), jnp.float32),
            pltpu.VMEM_SHARED((NPAD, D), jnp.float32),
            pltpu.SemaphoreType.DMA,
            pltpu.SemaphoreType.DMA,
            pltpu.SemaphoreType.DMA,
            pltpu.SemaphoreType.DMA,
        ],
        compiler_params=pltpu.CompilerParams(use_tc_tiling_on_sc=False),
    )
    def prop_kernel(table_hbm, srcs_hbm, dsts_hbm, ws_hbm, out_hbm,
                    src_v, dst_v, w_v, g0, g1, s0, s1, io_v, acc_sh,
                    tab_sh, sg0, sg1, ss0, ss1):
        cid = lax.axis_index("c")
        sid = lax.axis_index("s")
        wid = cid * NS + sid

        pltpu.sync_copy(srcs_hbm.at[wid], src_v)
        pltpu.sync_copy(dsts_hbm.at[wid], dst_v)
        pltpu.sync_copy(ws_hbm.at[wid], w_v)

        def zero_body(r, _):
            for k in range(D // 16):
                io_v[r, pl.ds(k * 16, 16)] = jnp.zeros((16,), jnp.float32)
            return 0
        lax.fori_loop(0, RB, zero_body, 0)
        for c in range(NB):
            pltpu.sync_copy(io_v, acc_sh.at[pl.ds(sid * RPT + c * RB, RB)])
        for c in range(NB):
            pltpu.sync_copy(table_hbm.at[pl.ds(sid * RPT + c * RB, RB)],
                            tab_sh.at[pl.ds(sid * RPT + c * RB, RB)])
        plsc.subcore_barrier()

        def issue_gather(j, buf, sem):
            pltpu.async_copy(tab_sh.at[src_v.at[j]], buf, sem)

        def wait_gather(j, buf, sem):
            pltpu.make_async_copy(tab_sh.at[src_v.at[j]], buf, sem).wait()

        def issue_scatter(j, buf, sem):
            pltpu.async_copy(buf, acc_sh.at[dst_v.at[j]], sem, add=True)

        def wait_scatter(j, buf, sem):
            pltpu.make_async_copy(buf, acc_sh.at[dst_v.at[j]], sem).wait()

        def scale(j, gb, sb):
            def blk(b, _):
                w16 = w_v[j, pl.ds(b * 16, 16)]
                for l in range(16):
                    e = b * 16 + l
                    wv = _lane_bcast(w16, l)
                    for k in range(D // 16):
                        sb[e, pl.ds(k * 16, 16)] = (
                            gb[e, pl.ds(k * 16, 16)] * wv)
                return 0
            lax.fori_loop(0, 8, blk, 0)

        nhalf = CH // 2
        issue_gather(0, g0, sg0)
        issue_gather(1, g1, sg1)

        def pair_body(i, _):
            for b, gb, sb, sg, ss in ((0, g0, s0, sg0, ss0),
                                      (1, g1, s1, sg1, ss1)):
                j = 2 * i + b
                wait_gather(j, gb, sg)

                @pl.when(i >= 1)
                def _(j=j, sb=sb, ss=ss):
                    wait_scatter(j - 2, sb, ss)

                scale(j, gb, sb)
                issue_scatter(j, sb, ss)

                @pl.when(i < nhalf - 1)
                def _(j=j, gb=gb, sg=sg):
                    issue_gather(j + 2, gb, sg)
            return 0
        lax.fori_loop(0, nhalf, pair_body, 0)
        wait_scatter(CH - 2, s0, ss0)
        wait_scatter(CH - 1, s1, ss1)
        plsc.subcore_barrier()

        for c in range(NB):
            pltpu.sync_copy(acc_sh.at[pl.ds(sid * RPT + c * RB, RB)], io_v)
            pltpu.sync_copy(io_v, out_hbm.at[cid, pl.ds(sid * RPT + c * RB, RB)])

    return prop_kernel


def _tc_call(body, out_shapes, *args):
    return pl.pallas_call(
        body,
        out_shape=out_shapes,
    )(*args)


def _stage_a(dg0_ref, dg1_ref, x4_ref, w1a_ref, w1b_ref,
             xs1a_ref, xs1b_ref, dise_ref, dis2e_ref):
    degw = dg0_ref[...] + dg1_ref[...]
    dis = lax.rsqrt(degw + 1.0)
    dis2 = jnp.where(degw > 0.0, lax.rsqrt(jnp.maximum(degw, 1e-30)), 0.0)
    x4 = x4_ref[...]
    xs1a_ref[...] = dis * jnp.dot(x4, w1a_ref[...],
                                  preferred_element_type=jnp.float32)
    xs1b_ref[...] = dis * jnp.dot(x4, w1b_ref[...],
                                  preferred_element_type=jnp.float32)
    dise_ref[...] = dis
    dis2e_ref[...] = dis2


def _sum_parts(s_ref):
    n = s_ref.shape[0] // 2
    return s_ref[:n, :] + s_ref[n:, :]


def _stage_b(sa_ref, sb_ref, xs1a_ref, xs1b_ref,
             dise_ref, w2a_ref, w2b_ref, b1a_ref, b1b_ref, xs2_ref):
    dise = dise_ref[...]
    h1a = jnp.maximum(
        dise * (_sum_parts(sa_ref) + xs1a_ref[...]) + b1a_ref[...], 0.0)
    h1b = jnp.maximum(
        dise * (_sum_parts(sb_ref) + xs1b_ref[...]) + b1b_ref[...], 0.0)
    xs2_ref[...] = dise * (
        jnp.dot(h1a, w2a_ref[...], preferred_element_type=jnp.float32)
        + jnp.dot(h1b, w2b_ref[...], preferred_element_type=jnp.float32))


def _stage_c(s2_ref, xs2_ref, dise_ref, dis2e_ref, b2_ref,
             h2_ref, xs3_ref):
    h2 = jnp.maximum(
        dise_ref[...] * (_sum_parts(s2_ref) + xs2_ref[...])
        + b2_ref[...], 0.0)
    h2_ref[...] = h2
    xs3_ref[...] = dis2e_ref[...] * h2


def _stage_d(s3_ref, h2_ref, dis2e_ref,
             w00_ref, w01_ref, b0_ref, w20_ref, w21_ref, b2_ref,
             w30_ref, w31_ref, b3_ref, wc2_ref, lw_ref, lb_ref, out_ref):
    h2 = h2_ref[...]
    tx1 = -dis2e_ref[...] * _sum_parts(s3_ref)
    g0 = (jnp.dot(h2, w00_ref[...], preferred_element_type=jnp.float32)
          + jnp.dot(tx1, w01_ref[...], preferred_element_type=jnp.float32)
          + b0_ref[...])
    g2 = (jnp.dot(h2, w20_ref[...], preferred_element_type=jnp.float32)
          + jnp.dot(tx1, w21_ref[...], preferred_element_type=jnp.float32)
          + b2_ref[...])
    g3 = (jnp.dot(h2, w30_ref[...], preferred_element_type=jnp.float32)
          + jnp.dot(tx1, w31_ref[...], preferred_element_type=jnp.float32)
          + b3_ref[...])
    gi = jax.nn.sigmoid(g0)
    gt = jnp.tanh(g2)
    c = gi * gt
    go = jax.nn.sigmoid(g3 + wc2_ref[...] * c)
    h = go * jnp.tanh(c)
    out_ref[...] = (jnp.dot(jnp.maximum(h, 0.0), lw_ref[...],
                            preferred_element_type=jnp.float32)
                    + lb_ref[...])


def kernel(x, edge_index, edge_attr, conv1_W, conv1_b, conv2_W, conv2_b,
           Wx, bx, Wh, bh, wc, bg, lin_W, lin_b):
    N, DIN = x.shape
    E = edge_index.shape[1]
    info = plsc.get_sparse_core_info()
    NC, NS = info.num_cores, info.num_subcores
    NW = NC * NS

    CH = -(-E // (NW * 128))
    CH = CH + (CH % 2)
    EPAD = NW * CH * 128
    RPT = -(-(N + 1) // NS)
    RPT = -(-RPT // 128) * 128
    NPAD = RPT * NS

    f32 = jnp.float32
    src = edge_index[0].astype(jnp.int32)
    dst = edge_index[1].astype(jnp.int32)
    pad = EPAD - E
    srcs = jnp.concatenate([src, jnp.zeros((pad,), jnp.int32)]).reshape(NW, CH, 128)
    dsts = jnp.concatenate([dst, jnp.full((pad,), N, jnp.int32)]).reshape(NW, CH, 128)
    ws = jnp.concatenate([edge_attr.astype(f32), jnp.zeros((pad,), f32)]).reshape(NW, CH, 128)
    xpad = jnp.pad(x.astype(f32), ((0, NPAD - N), (0, 0)))

    D1 = conv1_W.shape[1]
    D2 = conv2_W.shape[1]
    DH1 = D1 // 2
    PK = NPAD // 4

    def blk4(w):
        return jnp.kron(jnp.eye(4, dtype=f32), w.astype(f32))

    def tile4(b):
        return jnp.tile(b.astype(f32), 4).reshape(1, -1)

    pk = jax.ShapeDtypeStruct((PK, 128), f32)

    degp = _make_deg_kernel(NC, NS, NPAD, RPT, CH)(dsts, ws)
    dg0 = jnp.broadcast_to(degp[0].reshape(PK, 4, 1), (PK, 4, 32)).reshape(PK, 128)
    dg1 = jnp.broadcast_to(degp[1].reshape(PK, 4, 1), (PK, 4, 32)).reshape(PK, 128)

    w1 = conv1_W.astype(f32)
    xs1a_pk, xs1b_pk, dise, dis2e = _tc_call(
        _stage_a, (pk, pk, pk, pk),
        dg0, dg1, xpad.reshape(PK, 4 * DIN),
        blk4(w1[:, :DH1]), blk4(w1[:, DH1:]))

    prop32 = _make_prop_kernel(NC, NS, NPAD, RPT, CH, DH1)
    s1a = prop32(xs1a_pk.reshape(NPAD, DH1), srcs, dsts, ws).reshape(2 * PK, 128)
    s1b = prop32(xs1b_pk.reshape(NPAD, DH1), srcs, dsts, ws).reshape(2 * PK, 128)

    w2 = conv2_W.astype(f32)
    b1 = conv1_b.astype(f32)
    (xs2_pk,) = _tc_call(
        _stage_b, (pk,),
        s1a, s1b, xs1a_pk, xs1b_pk, dise,
        blk4(w2[:DH1]), blk4(w2[DH1:]),
        tile4(b1[:DH1]), tile4(b1[DH1:]))

    s2 = prop32(xs2_pk.reshape(NPAD, D2), srcs, dsts, ws).reshape(2 * PK, 128)

    h2_pk, xs3_pk = _tc_call(
        _stage_c, (pk, pk),
        s2, xs2_pk, dise, dis2e, tile4(conv2_b))

    s3 = prop32(xs3_pk.reshape(NPAD, D2), srcs, dsts, ws).reshape(2 * PK, 128)

    DG = Wx.shape[3]
    DO = lin_W.shape[1]
    bsum = (bx + bh + bg).astype(f32)
    (out_pk,) = _tc_call(
        _stage_d,
        (jax.ShapeDtypeStruct((PK, 4 * DO), f32),),
        s3, h2_pk, dis2e,
        blk4(Wx[0, 0]), blk4(Wx[0, 1]), tile4(bsum[0]),
        blk4(Wx[2, 0]), blk4(Wx[2, 1]), tile4(bsum[2]),
        blk4(Wx[3, 0]), blk4(Wx[3, 1]), tile4(bsum[3]),
        tile4(wc[2]), blk4(lin_W), tile4(lin_b))

    return out_pk.reshape(NPAD, DO)[:N]
